# baseline (speedup 1.0000x reference)
import functools

import jax
import jax.numpy as jnp
from jax import lax
from jax.experimental import pallas as pl
from jax.experimental.pallas import tpu as pltpu
from jax.experimental.pallas import tpu_sc as plsc

EMBD = 64
CHUNK = 128
NBUF = 6
BLK = 512


@functools.lru_cache(maxsize=None)
def _make_transpose(vocab: int):
    info = plsc.get_sparse_core_info()
    nc = info.num_cores
    nw = nc * info.num_subcores
    nblk = vocab // BLK
    tail = vocab - nblk * BLK
    tail_w = nw - 1
    assert tail % 16 == 0 and (nblk * BLK) % 128 == 0

    mesh = plsc.VectorSubcoreMesh(core_axis_name="c", subcore_axis_name="s")

    @functools.partial(
        pl.kernel,
        out_type=jax.ShapeDtypeStruct((vocab, EMBD), jnp.float32),
        mesh=mesh,
        scratch_types=[
            pltpu.VMEM((EMBD, BLK), jnp.float32),
            pltpu.VMEM((EMBD, BLK), jnp.float32),
            pltpu.VMEM((BLK, EMBD + 1), jnp.float32),
            pltpu.SemaphoreType.DMA((2,)),
            pltpu.SemaphoreType.DMA,
        ],
        compiler_params=pltpu.CompilerParams(
            use_tc_tiling_on_sc=False, needs_layout_passes=False
        ),
    )
    def transpose(tbl_t_hbm, scr_hbm, in0, in1, outb, isems, osem):
        wid = lax.axis_index("s") * nc + lax.axis_index("c")
        nb = jnp.where(wid < (nblk % nw), nblk // nw + 1, nblk // nw)
        lanes = lax.iota(jnp.int32, 16)

        dgroups = [g * 16 + lanes for g in range(EMBD // 16)]

        def do_transpose(buf, n16):
            def inner(c, _):
                cols = jnp.full((16,), 0, jnp.int32) + c
                for g in range(EMBD // 16):
                    vals = plsc.load_gather(buf, [dgroups[g], cols])
                    outb[c, pl.ds(g * 16, 16)] = vals
                return 0

            lax.fori_loop(0, n16 * 16, inner, 0)

        def in_src(t):
            g = wid + t * nw
            return tbl_t_hbm.at[:, pl.ds(g * BLK, BLK)]

        def start_in(t, buf):
            pltpu.async_copy(in_src(t), buf, isems.at[t % 2])

        def wait_in(t, buf):
            pltpu.make_async_copy(in_src(t), buf, isems.at[t % 2]).wait()

        def out_dst(t):
            g = wid + t * nw
            return scr_hbm.at[pl.ds(g * BLK, BLK)]

        def start_out(t):
            pltpu.async_copy(outb.at[:, pl.ds(0, EMBD)], out_dst(t), osem)

        def wait_out(t):
            pltpu.make_async_copy(
                outb.at[:, pl.ds(0, EMBD)], out_dst(t), osem
            ).wait()

        start_in(0, in0)

        def body(t, _):
            even = lax.rem(t, 2) == 0

            @pl.when(even)
            def _():
                wait_in(t, in0)

            @pl.when(jnp.logical_not(even))
            def _():
                wait_in(t, in1)

            @pl.when(t + 1 < nb)
            def _():
                @pl.when(even)
                def _():
                    start_in(t + 1, in1)

                @pl.when(jnp.logical_not(even))
                def _():
                    start_in(t + 1, in0)

            @pl.when(t > 0)
            def _():
                wait_out(t - 1)

            @pl.when(even)
            def _():
                do_transpose(in0, BLK // 16)

            @pl.when(jnp.logical_not(even))
            def _():
                do_transpose(in1, BLK // 16)

            start_out(t)
            return 0

        lax.fori_loop(0, nb, body, 0)
        wait_out(nb - 1)

        if tail:
            @pl.when(wid == tail_w)
            def _():
                base = nblk * BLK
                src = tbl_t_hbm.at[:, pl.ds(base, tail)]
                buf = in0.at[:, pl.ds(0, tail)]
                pltpu.async_copy(src, buf, isems.at[0]).wait()
                do_transpose(in0, tail // 16)
                pltpu.async_copy(
                    outb.at[pl.ds(0, tail), pl.ds(0, EMBD)],
                    scr_hbm.at[pl.ds(base, tail)],
                    osem,
                ).wait()

    return transpose


@functools.lru_cache(maxsize=None)
def _make_gather(n_rows: int, vocab: int):
    info = plsc.get_sparse_core_info()
    nw = info.num_cores * info.num_subcores
    assert n_rows % (nw * CHUNK) == 0
    chunks_per_w = n_rows // (nw * CHUNK)
    rows_per_w = chunks_per_w * CHUNK

    mesh = plsc.VectorSubcoreMesh(core_axis_name="c", subcore_axis_name="s")

    @functools.partial(
        pl.kernel,
        out_type=jax.ShapeDtypeStruct((n_rows, EMBD), jnp.float32),
        mesh=mesh,
        scratch_types=[
            pltpu.VMEM((chunks_per_w, CHUNK), jnp.int32),
            pltpu.VMEM((NBUF, CHUNK, EMBD), jnp.float32),
            pltpu.SemaphoreType.DMA((NBUF,)),
            pltpu.SemaphoreType.DMA((NBUF,)),
        ],
        compiler_params=pltpu.CompilerParams(use_tc_tiling_on_sc=False),
    )
    def gather(idx_hbm, scr_hbm, out_hbm, idx_v, rows_v, gsem, osem):
        wid = lax.axis_index("s") * info.num_cores + lax.axis_index("c")
        pltpu.sync_copy(idx_hbm.at[wid], idx_v)
        out_base = wid * rows_per_w

        def start_gather(k):
            b = lax.rem(k, NBUF)
            pltpu.async_copy(scr_hbm.at[idx_v.at[k]], rows_v.at[b], gsem.at[b])

        def wait_gather(k):
            b = lax.rem(k, NBUF)
            pltpu.make_async_copy(
                scr_hbm.at[idx_v.at[k]], rows_v.at[b], gsem.at[b]
            ).wait()

        def out_ref(k):
            return out_hbm.at[pl.ds(out_base + k * CHUNK, CHUNK)]

        def start_out(k):
            b = lax.rem(k, NBUF)
            pltpu.async_copy(rows_v.at[b], out_ref(k), osem.at[b])

        def wait_out(k):
            b = lax.rem(k, NBUF)
            pltpu.make_async_copy(rows_v.at[b], out_ref(k), osem.at[b]).wait()

        for k in range(NBUF - 1):
            start_gather(k)

        def body(j, _):
            jn = j + NBUF - 1

            @pl.when(jnp.logical_and(jn < chunks_per_w, j > 0))
            def _():
                wait_out(j - 1)

            @pl.when(jn < chunks_per_w)
            def _():
                start_gather(jn)

            wait_gather(j)
            start_out(j)
            return 0

        lax.fori_loop(0, chunks_per_w, body, 0)

        for t in range(NBUF):
            wait_out(chunks_per_w - NBUF + t)

    return gather


def kernel(x, table):
    batch, seq = x.shape
    n_rows = batch * seq
    vocab = table.shape[0]
    info = plsc.get_sparse_core_info()
    nw = info.num_cores * info.num_subcores
    tbl_t = jnp.swapaxes(table, 0, 1)
    idx = x.astype(jnp.int32).reshape(nw, n_rows // (nw * CHUNK), CHUNK)
    scr = _make_transpose(vocab)(tbl_t)
    out = _make_gather(n_rows, vocab)(idx, scr)
    return out.reshape(batch, seq, EMBD)


# speedup vs baseline: 8.9690x; 8.9690x over previous
import functools

import jax
import jax.numpy as jnp
from jax import lax
from jax.experimental import pallas as pl
from jax.experimental.pallas import tpu as pltpu
from jax.experimental.pallas import tpu_sc as plsc

EMBD = 64
CHUNK = 128
NBUF = 6


@functools.lru_cache(maxsize=None)
def _make_gather(n_rows: int):
    info = plsc.get_sparse_core_info()
    nw = info.num_cores * info.num_subcores
    assert n_rows % (nw * CHUNK) == 0
    chunks_per_w = n_rows // (nw * CHUNK)
    rows_per_w = chunks_per_w * CHUNK

    mesh = plsc.VectorSubcoreMesh(core_axis_name="c", subcore_axis_name="s")

    @functools.partial(
        pl.kernel,
        out_type=jax.ShapeDtypeStruct((n_rows, EMBD), jnp.float32),
        mesh=mesh,
        scratch_types=[
            pltpu.VMEM((chunks_per_w, CHUNK), jnp.int32),
            pltpu.VMEM((NBUF, CHUNK, EMBD), jnp.float32),
            pltpu.SemaphoreType.DMA((NBUF,)),
            pltpu.SemaphoreType.DMA((NBUF,)),
        ],
        compiler_params=pltpu.CompilerParams(use_tc_tiling_on_sc=False),
    )
    def gather(idx_hbm, table_hbm, out_hbm, idx_v, rows_v, gsem, osem):
        wid = lax.axis_index("s") * info.num_cores + lax.axis_index("c")
        pltpu.sync_copy(idx_hbm.at[wid], idx_v)
        out_base = wid * rows_per_w

        def start_gather(k):
            b = lax.rem(k, NBUF)
            pltpu.async_copy(table_hbm.at[idx_v.at[k]], rows_v.at[b], gsem.at[b])

        def wait_gather(k):
            b = lax.rem(k, NBUF)
            pltpu.make_async_copy(
                table_hbm.at[idx_v.at[k]], rows_v.at[b], gsem.at[b]
            ).wait()

        def out_ref(k):
            return out_hbm.at[pl.ds(out_base + k * CHUNK, CHUNK)]

        def start_out(k):
            b = lax.rem(k, NBUF)
            pltpu.async_copy(rows_v.at[b], out_ref(k), osem.at[b])

        def wait_out(k):
            b = lax.rem(k, NBUF)
            pltpu.make_async_copy(rows_v.at[b], out_ref(k), osem.at[b]).wait()

        for k in range(NBUF - 1):
            start_gather(k)

        def body(j, _):
            jn = j + NBUF - 1

            @pl.when(jnp.logical_and(jn < chunks_per_w, j > 0))
            def _():
                wait_out(j - 1)

            @pl.when(jn < chunks_per_w)
            def _():
                start_gather(jn)

            wait_gather(j)
            start_out(j)
            return 0

        lax.fori_loop(0, chunks_per_w, body, 0)

        for t in range(NBUF):
            wait_out(chunks_per_w - NBUF + t)

    return gather


def kernel(x, table):
    batch, seq = x.shape
    n_rows = batch * seq
    info = plsc.get_sparse_core_info()
    nw = info.num_cores * info.num_subcores
    x_t = jnp.swapaxes(x, 0, 1)
    idx = x_t.astype(jnp.int32).reshape(nw, n_rows // (nw * CHUNK), CHUNK)
    out = _make_gather(n_rows)(idx, table)
    return jnp.swapaxes(out.reshape(seq, batch, EMBD), 0, 1)
